# Initial kernel scaffold; baseline (speedup 1.0000x reference)
import jax
import jax.numpy as jnp
from jax.experimental import pallas as pl


def kernel(sparse_features, dense_features, tables, Wd, bd, W1, b1, W2, b2, W3, b3, Wo, bo):
    raise NotImplementedError("write your pallas kernel here")


# speedup vs baseline: 5.3837x; 5.3837x over previous
import functools

import jax
import jax.numpy as jnp
from jax import lax
from jax.experimental import pallas as pl
from jax.experimental.pallas import tpu as pltpu
from jax.experimental.pallas import tpu_sc as plsc

N_SPARSE_F = 26
VOCAB_SIZE = 100000
EMB_DIM = 16

SC_CORES = 2
SC_SUBCORES = 16
SC_WORKERS = SC_CORES * SC_SUBCORES

GATHER_CHUNK = 1664


def _sc_gather(flat_tables, idx):
    n_idx = idx.shape[0]
    per_worker = n_idx // SC_WORKERS
    n_chunks = per_worker // GATHER_CHUNK
    mesh = plsc.VectorSubcoreMesh(core_axis_name="c", subcore_axis_name="s")

    @functools.partial(
        pl.kernel,
        out_type=jax.ShapeDtypeStruct((n_idx, EMB_DIM), jnp.float32),
        mesh=mesh,
        compiler_params=pltpu.CompilerParams(use_tc_tiling_on_sc=False),
        scratch_types=[
            pltpu.VMEM((GATHER_CHUNK,), jnp.int32),
            pltpu.VMEM((GATHER_CHUNK, EMB_DIM), jnp.float32),
            pltpu.SemaphoreType.DMA,
        ],
    )
    def gather_kernel(table_hbm, idx_hbm, out_hbm, idx_v, rows_v, sem):
        wid = lax.axis_index("s") * SC_CORES + lax.axis_index("c")
        w_base = wid * per_worker

        @pl.loop(0, n_chunks)
        def _(t):
            base = w_base + t * GATHER_CHUNK
            pltpu.sync_copy(idx_hbm.at[pl.ds(base, GATHER_CHUNK)], idx_v)
            pltpu.async_copy(table_hbm.at[idx_v], rows_v, sem).wait()
            pltpu.sync_copy(rows_v, out_hbm.at[pl.ds(base, GATHER_CHUNK)])

    return gather_kernel(flat_tables, idx)


def _mlp_body(se_ref, df_ref, Wd_ref, bd_ref, W1s_ref, W1d_ref, b1_ref,
              W2_ref, b2_ref, W3_ref, b3_ref, Wo_ref, bo_ref, out_ref):
    se = se_ref[...]
    de = jnp.dot(df_ref[...], Wd_ref[...],
                 preferred_element_type=jnp.float32) + bd_ref[...]
    h = jnp.maximum(
        jnp.dot(se, W1s_ref[...], preferred_element_type=jnp.float32)
        + jnp.dot(de, W1d_ref[...], preferred_element_type=jnp.float32)
        + b1_ref[...], 0.0)
    h = jnp.maximum(
        jnp.dot(h, W2_ref[...], preferred_element_type=jnp.float32)
        + b2_ref[...], 0.0)
    h = jnp.maximum(
        jnp.dot(h, W3_ref[...], preferred_element_type=jnp.float32)
        + b3_ref[...], 0.0)
    fm = jnp.sum(se, axis=1) + jnp.sum(de, axis=1)
    logit = jnp.dot(h, Wo_ref[...], preferred_element_type=jnp.float32)[:, 0]
    out_ref[...] = logit + bo_ref[...] + 0.1 * fm


def _mlp(se, df, Wd, bd, W1s, W1d, b1, W2, b2, W3, b3, Wo, bo, tile_b=2048):
    B = se.shape[0]

    def full(a):
        return pl.BlockSpec(a.shape, lambda i: tuple(0 for _ in a.shape))

    return pl.pallas_call(
        _mlp_body,
        grid=(B // tile_b,),
        in_specs=[
            pl.BlockSpec((tile_b, se.shape[1]), lambda i: (i, 0)),
            pl.BlockSpec((tile_b, df.shape[1]), lambda i: (i, 0)),
            full(Wd), full(bd), full(W1s), full(W1d), full(b1),
            full(W2), full(b2), full(W3), full(b3), full(Wo), full(bo),
        ],
        out_specs=pl.BlockSpec((tile_b,), lambda i: (i,)),
        out_shape=jax.ShapeDtypeStruct((B,), jnp.float32),
    )(se, df, Wd, bd, W1s, W1d, b1, W2, b2, W3, b3, Wo, bo)


def kernel(sparse_features, dense_features, tables, Wd, bd, W1, b1, W2, b2,
           W3, b3, Wo, bo):
    B = sparse_features.shape[0]
    offs = jnp.arange(N_SPARSE_F, dtype=jnp.int32) * VOCAB_SIZE
    idx = (sparse_features.astype(jnp.int32) + offs[None, :]).reshape(-1)
    flat_tables = tables.reshape(N_SPARSE_F * VOCAB_SIZE, EMB_DIM)

    rows = _sc_gather(flat_tables, idx)
    se = rows.reshape(B, N_SPARSE_F * EMB_DIM)

    n_se = N_SPARSE_F * EMB_DIM
    W1s = W1[:n_se]
    W1d = W1[n_se:]
    return _mlp(se, dense_features, Wd, bd, W1s, W1d, b1, W2, b2, W3, b3,
                Wo, bo)
